# baseline (speedup 1.0000x reference)
import functools

import jax
import jax.numpy as jnp
from jax import lax
from jax.experimental import pallas as pl
from jax.experimental.pallas import tpu as pltpu
from jax.experimental.pallas import tpu_sc as plsc

_NC = 2
_NS = 16
_NW = _NC * _NS
_L = 16
_D = 64
_B = 16384
_BPW = _B // _NW
_CHUNK = 128
_NCHUNK = _BPW // _CHUNK


def _sc_body(user_hbm, item_hbm, uf_hbm, vf_hbm, out_hbm,
             idx_u, idx_v, rows_u, rows_v, p_buf, out_v, sem):
    wid = lax.axis_index("s") * _NC + lax.axis_index("c")
    base = wid * _BPW

    pltpu.sync_copy(user_hbm.at[pl.ds(base, _BPW)], idx_u)
    pltpu.sync_copy(item_hbm.at[pl.ds(base, _BPW)], idx_v)

    copies = []
    for j in range(_NCHUNK):
        s = pl.ds(j * _CHUNK, _CHUNK)
        copies.append(pltpu.async_copy(uf_hbm.at[idx_u.at[s]], rows_u.at[s], sem))
        copies.append(pltpu.async_copy(vf_hbm.at[idx_v.at[s]], rows_v.at[s], sem))
    for c in copies:
        c.wait()

    col_idx = lax.iota(jnp.int32, _L) * _L

    def group(g, _):
        for r in range(_L):
            row = g * _L + r
            acc = rows_u[row, pl.ds(0, _L)] * rows_v[row, pl.ds(0, _L)]
            for c in range(1, _D // _L):
                acc += rows_u[row, pl.ds(c * _L, _L)] * rows_v[row, pl.ds(c * _L, _L)]
            p_buf[pl.ds(r * _L, _L)] = acc
        tot = plsc.load_gather(p_buf, [col_idx])
        for j in range(1, _L):
            tot += plsc.load_gather(p_buf, [col_idx + j])
        out_v[pl.ds(g * _L, _L)] = tot
        return _

    lax.fori_loop(0, _BPW // _L, group, 0, unroll=False)

    pltpu.sync_copy(out_v, out_hbm.at[pl.ds(base, _BPW)])


@jax.jit
def kernel(user, item, user_factors, item_factors):
    mesh = plsc.VectorSubcoreMesh(core_axis_name="c", subcore_axis_name="s",
                                  num_cores=_NC, num_subcores=_NS)
    run = pl.kernel(
        _sc_body,
        out_type=jax.ShapeDtypeStruct((_B,), jnp.float32),
        mesh=mesh,
        scratch_types=[
            pltpu.VMEM((_BPW,), jnp.int32),
            pltpu.VMEM((_BPW,), jnp.int32),
            pltpu.VMEM((_BPW, _D), jnp.float32),
            pltpu.VMEM((_BPW, _D), jnp.float32),
            pltpu.VMEM((_L * _L,), jnp.float32),
            pltpu.VMEM((_BPW,), jnp.float32),
            pltpu.SemaphoreType.DMA,
        ],
        compiler_params=pltpu.CompilerParams(needs_layout_passes=False,
                                             use_tc_tiling_on_sc=False),
    )
    return run(user, item, user_factors, item_factors)


# speedup vs baseline: 2.5967x; 2.5967x over previous
import jax
import jax.numpy as jnp
from jax import lax
from jax.experimental import pallas as pl
from jax.experimental.pallas import tpu as pltpu
from jax.experimental.pallas import tpu_sc as plsc

_NC = 2
_NS = 16
_NW = _NC * _NS
_L = 16
_D = 64
_B = 16384
_BPW = _B // _NW
_DEPTH = 4


_TAIL = 999936
_LASTC = 999808


def _sc_body(user_hbm, item_hbm, uT_hbm, vT_hbm, tu_hbm, tv_hbm, out_hbm,
             idx_u, idx_v, bu, bv, tail_u, tail_v, p_buf, out_v,
             s0, s1, s2, s3):
    wid = lax.axis_index("s") * _NC + lax.axis_index("c")
    base = wid * _BPW
    sems = [s0, s1, s2, s3]

    pltpu.sync_copy(user_hbm.at[pl.ds(base, _BPW)], idx_u)
    pltpu.sync_copy(item_hbm.at[pl.ds(base, _BPW)], idx_v)
    pltpu.sync_copy(tu_hbm, tail_u)
    pltpu.sync_copy(tv_hbm, tail_v)

    def issue(iu, iv, slot, sem):
        cu = pl.multiple_of(
            jnp.minimum(lax.shift_right_logical(iu, 7) * 128, _LASTC), 128)
        cv = pl.multiple_of(
            jnp.minimum(lax.shift_right_logical(iv, 7) * 128, _LASTC), 128)
        pltpu.async_copy(uT_hbm.at[:, pl.ds(cu, 128)], bu.at[slot], sem)
        pltpu.async_copy(vT_hbm.at[:, pl.ds(cv, 128)], bv.at[slot], sem)

    def drain(slot, sem):
        pltpu.make_async_copy(uT_hbm.at[:, pl.ds(0, 128)], bu.at[slot], sem).wait()
        pltpu.make_async_copy(vT_hbm.at[:, pl.ds(0, 128)], bv.at[slot], sem).wait()

    lane = lax.iota(jnp.int32, _L)
    col_idx = lane * _L
    d_vecs = [lane + g * _L for g in range(_D // _L)]

    iu0 = idx_u[pl.ds(0, _L)]
    iv0 = idx_v[pl.ds(0, _L)]
    for k in range(_DEPTH):
        issue(iu0[k], iv0[k], k, sems[k])

    def step(j, carry):
        p = j * _L
        iu_vec = idx_u[pl.ds(p, _L)]
        iv_vec = idx_v[pl.ds(p, _L)]
        pn = jnp.minimum(p + _L, _BPW - _L)
        iu_nxt = idx_u[pl.ds(pn, _L)]
        iv_nxt = idx_v[pl.ds(pn, _L)]
        for k in range(_L):
            slot = k % _DEPTH
            drain(slot, sems[slot])
            iu = iu_vec[k]
            iv = iv_vec[k]
            lu = jnp.full((_L,), 0, jnp.int32) + lax.bitwise_and(iu, 127)
            lv = jnp.full((_L,), 0, jnp.int32) + lax.bitwise_and(iv, 127)
            tmu = jnp.full((_L,), iu >= _TAIL)
            tmv = jnp.full((_L,), iv >= _TAIL)
            tlu = jnp.full((_L,), 0, jnp.int32) + jnp.maximum(iu - _TAIL, 0)
            tlv = jnp.full((_L,), 0, jnp.int32) + jnp.maximum(iv - _TAIL, 0)
            acc = jnp.zeros((_L,), jnp.float32)
            for g in range(_D // _L):
                uvals = jnp.where(tmu,
                                  plsc.load_gather(tail_u, [d_vecs[g], tlu]),
                                  plsc.load_gather(bu.at[slot], [d_vecs[g], lu]))
                vvals = jnp.where(tmv,
                                  plsc.load_gather(tail_v, [d_vecs[g], tlv]),
                                  plsc.load_gather(bv.at[slot], [d_vecs[g], lv]))
                acc += uvals * vvals
            p_buf[pl.ds(k * _L, _L)] = acc
            ahead = k + _DEPTH
            @pl.when(p + ahead < _BPW)
            def _():
                if ahead < _L:
                    issue(iu_vec[ahead], iv_vec[ahead], slot, sems[slot])
                else:
                    issue(iu_nxt[ahead - _L], iv_nxt[ahead - _L], slot, sems[slot])
        tot = plsc.load_gather(p_buf, [col_idx])
        for t in range(1, _L):
            tot += plsc.load_gather(p_buf, [col_idx + t])
        out_v[pl.ds(p, _L)] = tot
        return carry

    lax.fori_loop(0, _BPW // _L, step, 0)

    pltpu.sync_copy(out_v, out_hbm.at[pl.ds(base, _BPW)])


@jax.jit
def kernel(user, item, user_factors, item_factors):
    uT = user_factors.T
    vT = item_factors.T
    tu = user_factors[_TAIL:].T
    tv = item_factors[_TAIL:].T
    mesh = plsc.VectorSubcoreMesh(core_axis_name="c", subcore_axis_name="s",
                                  num_cores=_NC, num_subcores=_NS)
    run = pl.kernel(
        _sc_body,
        out_type=jax.ShapeDtypeStruct((_B,), jnp.float32),
        mesh=mesh,
        scratch_types=[
            pltpu.VMEM((_BPW,), jnp.int32),
            pltpu.VMEM((_BPW,), jnp.int32),
            pltpu.VMEM((_DEPTH, _D, 128), jnp.float32),
            pltpu.VMEM((_DEPTH, _D, 128), jnp.float32),
            pltpu.VMEM((_D, _D), jnp.float32),
            pltpu.VMEM((_D, _D), jnp.float32),
            pltpu.VMEM((_L * _L,), jnp.float32),
            pltpu.VMEM((_BPW,), jnp.float32),
            pltpu.SemaphoreType.DMA,
            pltpu.SemaphoreType.DMA,
            pltpu.SemaphoreType.DMA,
            pltpu.SemaphoreType.DMA,
        ],
        compiler_params=pltpu.CompilerParams(needs_layout_passes=False,
                                             use_tc_tiling_on_sc=True),
    )
    return run(user, item, uT, vT, tu, tv)


# speedup vs baseline: 2.8268x; 1.0886x over previous
import jax
import jax.numpy as jnp
from jax import lax
from jax.experimental import pallas as pl
from jax.experimental.pallas import tpu as pltpu
from jax.experimental.pallas import tpu_sc as plsc

_NC = 2
_NS = 16
_NW = _NC * _NS
_L = 16
_D = 64
_B = 16384
_BPW = _B // _NW
_DEPTH = 6


_TAIL = 999936
_LASTC = 999808


def _sc_body(user_hbm, item_hbm, uT_hbm, vT_hbm, tu_hbm, tv_hbm, out_hbm,
             idx_u, idx_v, bu, bv, tail_u, tail_v, p_buf, out_v,
             s0, s1, s2, s3, s4, s5):
    wid = lax.axis_index("s") * _NC + lax.axis_index("c")
    base = wid * _BPW
    sems = [s0, s1, s2, s3, s4, s5]

    pltpu.sync_copy(user_hbm.at[pl.ds(base, _BPW)], idx_u)
    pltpu.sync_copy(item_hbm.at[pl.ds(base, _BPW)], idx_v)
    pltpu.sync_copy(tu_hbm, tail_u)
    pltpu.sync_copy(tv_hbm, tail_v)

    def issue(iu, iv, slot, sem):
        cu = pl.multiple_of(
            jnp.minimum(lax.shift_right_logical(iu, 7) * 128, _LASTC), 128)
        cv = pl.multiple_of(
            jnp.minimum(lax.shift_right_logical(iv, 7) * 128, _LASTC), 128)
        pltpu.async_copy(uT_hbm.at[:, pl.ds(cu, 128)], bu.at[slot], sem)
        pltpu.async_copy(vT_hbm.at[:, pl.ds(cv, 128)], bv.at[slot], sem)

    def drain(slot, sem):
        pltpu.make_async_copy(uT_hbm.at[:, pl.ds(0, 128)], bu.at[slot], sem).wait()
        pltpu.make_async_copy(vT_hbm.at[:, pl.ds(0, 128)], bv.at[slot], sem).wait()

    lane = lax.iota(jnp.int32, _L)
    col_idx = lane * _L
    d_vecs = [lane + g * _L for g in range(_D // _L)]

    iu0 = idx_u[pl.ds(0, _L)]
    iv0 = idx_v[pl.ds(0, _L)]
    for k in range(_DEPTH):
        issue(iu0[k], iv0[k], k, sems[k])

    def step(j, carry):
        p = j * _L
        iu_vec = idx_u[pl.ds(p, _L)]
        iv_vec = idx_v[pl.ds(p, _L)]
        pn = jnp.minimum(p + _L, _BPW - _L)
        iu_nxt = idx_u[pl.ds(pn, _L)]
        iv_nxt = idx_v[pl.ds(pn, _L)]
        for k in range(_L):
            slot = k % _DEPTH
            drain(slot, sems[slot])
            iu = iu_vec[k]
            iv = iv_vec[k]
            lu = jnp.full((_L,), 0, jnp.int32) + lax.bitwise_and(iu, 127)
            lv = jnp.full((_L,), 0, jnp.int32) + lax.bitwise_and(iv, 127)
            tmu = jnp.full((_L,), iu >= _TAIL)
            tmv = jnp.full((_L,), iv >= _TAIL)
            tlu = jnp.full((_L,), 0, jnp.int32) + jnp.maximum(iu - _TAIL, 0)
            tlv = jnp.full((_L,), 0, jnp.int32) + jnp.maximum(iv - _TAIL, 0)
            acc = jnp.zeros((_L,), jnp.float32)
            for g in range(_D // _L):
                uvals = jnp.where(tmu,
                                  plsc.load_gather(tail_u, [d_vecs[g], tlu]),
                                  plsc.load_gather(bu.at[slot], [d_vecs[g], lu]))
                vvals = jnp.where(tmv,
                                  plsc.load_gather(tail_v, [d_vecs[g], tlv]),
                                  plsc.load_gather(bv.at[slot], [d_vecs[g], lv]))
                acc += uvals * vvals
            p_buf[pl.ds(k * _L, _L)] = acc
            ahead = k + _DEPTH
            @pl.when(p + ahead < _BPW)
            def _():
                if ahead < _L:
                    issue(iu_vec[ahead], iv_vec[ahead], slot, sems[slot])
                else:
                    issue(iu_nxt[ahead - _L], iv_nxt[ahead - _L], slot, sems[slot])
        tot = plsc.load_gather(p_buf, [col_idx])
        for t in range(1, _L):
            tot += plsc.load_gather(p_buf, [col_idx + t])
        out_v[pl.ds(p, _L)] = tot
        return carry

    lax.fori_loop(0, _BPW // _L, step, 0)

    pltpu.sync_copy(out_v, out_hbm.at[pl.ds(base, _BPW)])


@jax.jit
def kernel(user, item, user_factors, item_factors):
    uT = user_factors.T
    vT = item_factors.T
    tu = user_factors[_TAIL:].T
    tv = item_factors[_TAIL:].T
    mesh = plsc.VectorSubcoreMesh(core_axis_name="c", subcore_axis_name="s",
                                  num_cores=_NC, num_subcores=_NS)
    run = pl.kernel(
        _sc_body,
        out_type=jax.ShapeDtypeStruct((_B,), jnp.float32),
        mesh=mesh,
        scratch_types=[
            pltpu.VMEM((_BPW,), jnp.int32),
            pltpu.VMEM((_BPW,), jnp.int32),
            pltpu.VMEM((_DEPTH, _D, 128), jnp.float32),
            pltpu.VMEM((_DEPTH, _D, 128), jnp.float32),
            pltpu.VMEM((_D, _D), jnp.float32),
            pltpu.VMEM((_D, _D), jnp.float32),
            pltpu.VMEM((_L * _L,), jnp.float32),
            pltpu.VMEM((_BPW,), jnp.float32),
            pltpu.SemaphoreType.DMA,
            pltpu.SemaphoreType.DMA,
            pltpu.SemaphoreType.DMA,
            pltpu.SemaphoreType.DMA,
            pltpu.SemaphoreType.DMA,
            pltpu.SemaphoreType.DMA,
        ],
        compiler_params=pltpu.CompilerParams(needs_layout_passes=False,
                                             use_tc_tiling_on_sc=True),
    )
    return run(user, item, uT, vT, tu, tv)
